# baseline (speedup 1.0000x reference)
import jax
import jax.numpy as jnp
from jax import lax
from jax.experimental import pallas as pl
from jax.experimental.pallas import tpu as pltpu
from jax.experimental.pallas import tpu_sc as plsc

N = 10000
E = 320000
D_IN = 128
D_HID = 64

NC = 2
NS = 16
NW = NC * NS
L = 16

NPAD = 10240
DPT = D_HID // NS
NPK = DPT // 2
EPT = E // NW
EHALF = E // NC
CHUNK = 3200

_mesh = lambda: plsc.VectorSubcoreMesh(core_axis_name="c", subcore_axis_name="s")


def _zero_vmem(ref, total):
    z = jnp.zeros((L,), jnp.float32)

    def body(i, _):
        ref[pl.ds(i * L, L)] = z
        return 0

    lax.fori_loop(0, total // L, body, 0)


def _zero_vmem2(ref, rows, cols):
    z = jnp.zeros((L,), jnp.float32)

    def body(i, _):
        for j in range(rows):
            ref[j, pl.ds(i * L, L)] = z
        return 0

    lax.fori_loop(0, cols // L, body, 0)


def _sc_deg_body(col_hbm, ew_hbm, degp_hbm, cbuf, wbuf, acc):
    cid = lax.axis_index("c")
    sid = lax.axis_index("s")
    wid = cid * NS + sid
    pltpu.sync_copy(col_hbm.at[pl.ds(wid * EPT, EPT)], cbuf)
    pltpu.sync_copy(ew_hbm.at[pl.ds(wid * EPT, EPT)], wbuf)
    _zero_vmem(acc, NPAD)

    @plsc.parallel_loop(0, EPT, step=L, unroll=4)
    def body(i):
        c = cbuf[pl.ds(i, L)]
        w = wbuf[pl.ds(i, L)]
        plsc.addupdate_scatter(acc, [c], w)

    pltpu.sync_copy(acc, degp_hbm.at[wid])


def _sc_deg(col, ew):
    k = pl.kernel(
        _sc_deg_body,
        out_type=jax.ShapeDtypeStruct((NW, NPAD), jnp.float32),
        mesh=_mesh(),
        compiler_params=pltpu.CompilerParams(needs_layout_passes=False),
        scratch_types=[
            pltpu.VMEM((EPT,), jnp.int32),
            pltpu.VMEM((EPT,), jnp.float32),
            pltpu.VMEM((NPAD,), jnp.float32),
        ],
    )
    return k(col, ew)


def _tc1_body(x_ref, w1_ref, degp_ref, xwT_ref, xwp_ref, dis_ref):
    xb = x_ref[...]
    w = w1_ref[...]
    mm = lax.dot_general(
        w, xb, (((0,), (1,)), ((), ())), preferred_element_type=jnp.float32
    )
    deg = jnp.sum(degp_ref[...], axis=0) + 1.0
    dis = lax.rsqrt(deg)
    dis_ref[...] = dis
    xwT = mm * dis[None, :]
    xwT_ref[...] = xwT
    bf = lax.convert_element_type(xwT, jnp.bfloat16)
    bits = lax.convert_element_type(
        lax.bitcast_convert_type(bf, jnp.uint16), jnp.uint32
    )
    pairs = bits.reshape(D_HID // 2, 2, bits.shape[-1])
    packed = (pairs[:, 1, :] << 16) | pairs[:, 0, :]
    xwp_ref[...] = lax.bitcast_convert_type(packed, jnp.float32)


def _tc1(xp, W1, degp):
    B = 2048
    grid = NPAD // B
    return pl.pallas_call(
        _tc1_body,
        grid=(grid,),
        in_specs=[
            pl.BlockSpec((B, D_IN), lambda j: (j, 0)),
            pl.BlockSpec((D_IN, D_HID), lambda j: (0, 0)),
            pl.BlockSpec((NW, B), lambda j: (0, j)),
        ],
        out_specs=[
            pl.BlockSpec((D_HID, B), lambda j: (0, j)),
            pl.BlockSpec((D_HID // 2, B), lambda j: (0, j)),
            pl.BlockSpec((B,), lambda j: (j,)),
        ],
        out_shape=[
            jax.ShapeDtypeStruct((D_HID, NPAD), jnp.float32),
            jax.ShapeDtypeStruct((D_HID // 2, NPAD), jnp.float32),
            jax.ShapeDtypeStruct((NPAD,), jnp.float32),
        ],
    )(xp, W1, degp)


def _sc_agg64_body(xwp_hbm, ei_hbm, ew_hbm, p_hbm,
                   xw0, xw1, acc0, acc1, acc2, acc3,
                   rc0, rc1, wb0, wb1, sem0, sem1):
    cid = lax.axis_index("c")
    sid = lax.axis_index("s")
    base = cid * EHALF
    NCH = EHALF // CHUNK
    xws = [xw0, xw1]
    accs = [acc0, acc1, acc2, acc3]

    def start(g, rc, wb, sem):
        off = base + g * CHUNK
        pltpu.async_copy(ei_hbm.at[:, pl.ds(off, CHUNK)], rc, sem)
        pltpu.async_copy(ew_hbm.at[pl.ds(off, CHUNK)], wb, sem)

    def wait(rc, wb, sem):
        pltpu.make_async_copy(ei_hbm.at[:, pl.ds(0, CHUNK)], rc, sem).wait()
        pltpu.make_async_copy(ew_hbm.at[pl.ds(0, CHUNK)], wb, sem).wait()

    def process(rc, wb):
        @plsc.parallel_loop(0, CHUNK, step=L, unroll=4)
        def body(i):
            r = rc[0, pl.ds(i, L)]
            c = rc[1, pl.ds(i, L)]
            w = wb[pl.ds(i, L)]
            for k in range(NPK):
                g = plsc.load_gather(xws[k], [r])
                ab = plsc.bitcast(g, jnp.bfloat16)
                va, vb = plsc.unpack(
                    ab, format=plsc.PackFormat.INTERLEAVED,
                    preferred_element_type=jnp.float32,
                )
                plsc.addupdate_scatter(accs[2 * k], [c], va * w)
                plsc.addupdate_scatter(accs[2 * k + 1], [c], vb * w)

    start(0, rc0, wb0, sem0)
    for k in range(NPK):
        pltpu.sync_copy(xwp_hbm.at[sid * NPK + k], xws[k])
    for j in range(DPT):
        _zero_vmem(accs[j], NPAD)

    def outer(gg, _):
        g0 = gg * 2
        start(g0 + 1, rc1, wb1, sem1)
        wait(rc0, wb0, sem0)
        process(rc0, wb0)

        @pl.when(g0 + 2 < NCH)
        def _():
            start(g0 + 2, rc0, wb0, sem0)

        wait(rc1, wb1, sem1)
        process(rc1, wb1)
        return 0

    lax.fori_loop(0, NCH // 2, outer, 0)
    for j in range(DPT):
        pltpu.sync_copy(accs[j], p_hbm.at[cid, sid * DPT + j])


def _sc_agg64(xwp, ei, ew):
    k = pl.kernel(
        _sc_agg64_body,
        out_type=jax.ShapeDtypeStruct((NC, D_HID, NPAD), jnp.float32),
        mesh=_mesh(),
        compiler_params=pltpu.CompilerParams(needs_layout_passes=False),
        scratch_types=[
            pltpu.VMEM((NPAD,), jnp.float32),
            pltpu.VMEM((NPAD,), jnp.float32),
            pltpu.VMEM((NPAD,), jnp.float32),
            pltpu.VMEM((NPAD,), jnp.float32),
            pltpu.VMEM((NPAD,), jnp.float32),
            pltpu.VMEM((NPAD,), jnp.float32),
            pltpu.VMEM((2, CHUNK), jnp.int32),
            pltpu.VMEM((2, CHUNK), jnp.int32),
            pltpu.VMEM((CHUNK,), jnp.float32),
            pltpu.VMEM((CHUNK,), jnp.float32),
            pltpu.SemaphoreType.DMA,
            pltpu.SemaphoreType.DMA,
        ],
    )
    return k(xwp, ei, ew)


def _tc2_body(p_ref, xwT_ref, dis_ref, b1_ref, w2_ref, hw2_ref):
    dis = dis_ref[...]
    s = (p_ref[0] + p_ref[1] + xwT_ref[...]) * dis[None, :] + b1_ref[...]
    h = jnp.where(s > 0, s, jnp.exp(jnp.minimum(s, 0.0)) - 1.0)
    hw2_ref[...] = jnp.sum(h * w2_ref[...], axis=0) * dis


def _tc2(p, xwT, dis, b1c, W2):
    B = 2048
    grid = NPAD // B
    return pl.pallas_call(
        _tc2_body,
        grid=(grid,),
        in_specs=[
            pl.BlockSpec((NC, D_HID, B), lambda j: (0, 0, j)),
            pl.BlockSpec((D_HID, B), lambda j: (0, j)),
            pl.BlockSpec((B,), lambda j: (j,)),
            pl.BlockSpec((D_HID, 1), lambda j: (0, 0)),
            pl.BlockSpec((D_HID, 1), lambda j: (0, 0)),
        ],
        out_specs=pl.BlockSpec((B,), lambda j: (j,)),
        out_shape=jax.ShapeDtypeStruct((NPAD,), jnp.float32),
    )(p, xwT, dis, b1c, W2)


def _sc_agg1_body(hw2_hbm, row_hbm, col_hbm, ew_hbm, q_hbm,
                  hv, acc, rbuf, cbuf, wbuf):
    cid = lax.axis_index("c")
    sid = lax.axis_index("s")
    wid = cid * NS + sid
    pltpu.sync_copy(hw2_hbm, hv)
    pltpu.sync_copy(row_hbm.at[pl.ds(wid * EPT, EPT)], rbuf)
    pltpu.sync_copy(col_hbm.at[pl.ds(wid * EPT, EPT)], cbuf)
    pltpu.sync_copy(ew_hbm.at[pl.ds(wid * EPT, EPT)], wbuf)
    _zero_vmem(acc, NPAD)

    @plsc.parallel_loop(0, EPT, step=L, unroll=4)
    def body(i):
        r = rbuf[pl.ds(i, L)]
        c = cbuf[pl.ds(i, L)]
        w = wbuf[pl.ds(i, L)]
        hr = plsc.load_gather(hv, [r])
        plsc.addupdate_scatter(acc, [c], hr * w)

    pltpu.sync_copy(acc, q_hbm.at[wid])


def _sc_agg1(hw2, row, col, ew):
    k = pl.kernel(
        _sc_agg1_body,
        out_type=jax.ShapeDtypeStruct((NW, NPAD), jnp.float32),
        mesh=_mesh(),
        compiler_params=pltpu.CompilerParams(needs_layout_passes=False),
        scratch_types=[
            pltpu.VMEM((NPAD,), jnp.float32),
            pltpu.VMEM((NPAD,), jnp.float32),
            pltpu.VMEM((EPT,), jnp.int32),
            pltpu.VMEM((EPT,), jnp.int32),
            pltpu.VMEM((EPT,), jnp.float32),
        ],
    )
    return k(hw2, row, col, ew)


def _tc3_body(q_ref, hw2_ref, dis_ref, b2_ref, out_ref):
    z = (jnp.sum(q_ref[...], axis=0) + hw2_ref[...]) * dis_ref[...] + b2_ref[0, 0]
    out_ref[...] = 1.0 / (1.0 + jnp.exp(-z))


def _tc3(q, hw2, dis, b2c):
    B = 2048
    grid = NPAD // B
    return pl.pallas_call(
        _tc3_body,
        grid=(grid,),
        in_specs=[
            pl.BlockSpec((NW, B), lambda j: (0, j)),
            pl.BlockSpec((B,), lambda j: (j,)),
            pl.BlockSpec((B,), lambda j: (j,)),
            pl.BlockSpec((1, 1), lambda j: (0, 0)),
        ],
        out_specs=pl.BlockSpec((B,), lambda j: (j,)),
        out_shape=jax.ShapeDtypeStruct((NPAD,), jnp.float32),
    )(q, hw2, dis, b2c)


@jax.jit
def kernel(x, edge_index, edge_weight, W1, b1, W2, b2):
    row = edge_index[0]
    col = edge_index[1]
    xp = jnp.zeros((NPAD, D_IN), jnp.float32).at[:N].set(x)
    b1c = b1.reshape(D_HID, 1)
    b2c = b2.reshape(1, 1)

    degp = _sc_deg(col, edge_weight)
    xwT, xwp, dis = _tc1(xp, W1, degp)
    p = _sc_agg64(xwp, edge_index, edge_weight)
    hw2 = _tc2(p, xwT, dis, b1c, W2)
    q = _sc_agg1(hw2, row, col, edge_weight)
    out = _tc3(q, hw2, dis, b2c)
    return out[:N].reshape(N, 1)


# speedup vs baseline: 1.0489x; 1.0489x over previous
import jax
import jax.numpy as jnp
from jax import lax
from jax.experimental import pallas as pl
from jax.experimental.pallas import tpu as pltpu
from jax.experimental.pallas import tpu_sc as plsc

N = 10000
E = 320000
D_IN = 128
D_HID = 64

NC = 2
NS = 16
NW = NC * NS
L = 16

NPAD = 10240
DPT = D_HID // NS
NPK = DPT // 2
EPT = E // NW
EHALF = E // NC
CHUNK = 3200

_mesh = lambda: plsc.VectorSubcoreMesh(core_axis_name="c", subcore_axis_name="s")


def _zero_vmem(ref, total):
    z = jnp.zeros((L,), jnp.float32)

    def body(i, _):
        ref[pl.ds(i * L, L)] = z
        return 0

    lax.fori_loop(0, total // L, body, 0)


def _zero_vmem2(ref, rows, cols):
    z = jnp.zeros((L,), jnp.float32)

    def body(i, _):
        for j in range(rows):
            ref[j, pl.ds(i * L, L)] = z
        return 0

    lax.fori_loop(0, cols // L, body, 0)


def _sc_deg_body(col_hbm, ew_hbm, degp_hbm, cbuf, wbuf, acc):
    cid = lax.axis_index("c")
    sid = lax.axis_index("s")
    wid = cid * NS + sid
    pltpu.sync_copy(col_hbm.at[pl.ds(wid * EPT, EPT)], cbuf)
    pltpu.sync_copy(ew_hbm.at[pl.ds(wid * EPT, EPT)], wbuf)
    _zero_vmem(acc, NPAD)

    @plsc.parallel_loop(0, EPT, step=L, unroll=4)
    def body(i):
        c = cbuf[pl.ds(i, L)]
        w = wbuf[pl.ds(i, L)]
        plsc.addupdate_scatter(acc, [c], w)

    pltpu.sync_copy(acc, degp_hbm.at[wid])


def _sc_deg(col, ew):
    k = pl.kernel(
        _sc_deg_body,
        out_type=jax.ShapeDtypeStruct((NW, NPAD), jnp.float32),
        mesh=_mesh(),
        compiler_params=pltpu.CompilerParams(needs_layout_passes=False),
        scratch_types=[
            pltpu.VMEM((EPT,), jnp.int32),
            pltpu.VMEM((EPT,), jnp.float32),
            pltpu.VMEM((NPAD,), jnp.float32),
        ],
    )
    return k(col, ew)


def _tc1_body(x_ref, w1_ref, degp_ref, xwT_ref, xwp_ref, dis_ref):
    xb = x_ref[...]
    w = w1_ref[...]
    mm = lax.dot_general(
        w, xb, (((0,), (1,)), ((), ())), preferred_element_type=jnp.float32
    )
    deg = jnp.sum(degp_ref[...], axis=0) + 1.0
    dis = lax.rsqrt(deg)
    dis_ref[...] = dis
    xwT = mm * dis[None, :]
    xwT_ref[...] = xwT
    bf = lax.convert_element_type(xwT, jnp.bfloat16)
    bits = lax.convert_element_type(
        lax.bitcast_convert_type(bf, jnp.uint16), jnp.uint32
    )
    pairs = bits.reshape(D_HID // 2, 2, bits.shape[-1])
    packed = (pairs[:, 1, :] << 16) | pairs[:, 0, :]
    xwp_ref[...] = lax.bitcast_convert_type(packed, jnp.float32)


def _tc1(xp, W1, degp):
    B = 2048
    grid = NPAD // B
    return pl.pallas_call(
        _tc1_body,
        grid=(grid,),
        in_specs=[
            pl.BlockSpec((B, D_IN), lambda j: (j, 0)),
            pl.BlockSpec((D_IN, D_HID), lambda j: (0, 0)),
            pl.BlockSpec((NW, B), lambda j: (0, j)),
        ],
        out_specs=[
            pl.BlockSpec((D_HID, B), lambda j: (0, j)),
            pl.BlockSpec((D_HID // 2, B), lambda j: (0, j)),
            pl.BlockSpec((B,), lambda j: (j,)),
        ],
        out_shape=[
            jax.ShapeDtypeStruct((D_HID, NPAD), jnp.float32),
            jax.ShapeDtypeStruct((D_HID // 2, NPAD), jnp.float32),
            jax.ShapeDtypeStruct((NPAD,), jnp.float32),
        ],
    )(xp, W1, degp)


def _sc_agg64_body(xwp_hbm, ei_hbm, ew_hbm, p_hbm,
                   xwc, acc, rc0, rc1, wb0, wb1, sem0, sem1):
    cid = lax.axis_index("c")
    sid = lax.axis_index("s")
    base = cid * EHALF
    NCH = EHALF // CHUNK
    kvs = [jnp.full((L,), k, jnp.int32) for k in range(NPK)]
    jvs = [jnp.full((L,), j, jnp.int32) for j in range(DPT)]

    def start(g, rc, wb, sem):
        off = base + g * CHUNK
        pltpu.async_copy(ei_hbm.at[:, pl.ds(off, CHUNK)], rc, sem)
        pltpu.async_copy(ew_hbm.at[pl.ds(off, CHUNK)], wb, sem)

    def wait(rc, wb, sem):
        pltpu.make_async_copy(ei_hbm.at[:, pl.ds(0, CHUNK)], rc, sem).wait()
        pltpu.make_async_copy(ew_hbm.at[pl.ds(0, CHUNK)], wb, sem).wait()

    def process(rc, wb):
        @plsc.parallel_loop(0, CHUNK, step=L, unroll=4)
        def body(i):
            r = rc[0, pl.ds(i, L)]
            c = rc[1, pl.ds(i, L)]
            w = wb[pl.ds(i, L)]
            for k in range(NPK):
                g = plsc.load_gather(xwc, [kvs[k], r])
                ab = plsc.bitcast(g, jnp.bfloat16)
                va, vb = plsc.unpack(
                    ab, format=plsc.PackFormat.INTERLEAVED,
                    preferred_element_type=jnp.float32,
                )
                plsc.addupdate_scatter(acc, [jvs[2 * k], c], va * w)
                plsc.addupdate_scatter(acc, [jvs[2 * k + 1], c], vb * w)

    start(0, rc0, wb0, sem0)
    pltpu.sync_copy(xwp_hbm.at[pl.ds(sid * NPK, NPK)], xwc)
    _zero_vmem2(acc, DPT, NPAD)

    def outer(gg, _):
        g0 = gg * 2
        start(g0 + 1, rc1, wb1, sem1)
        wait(rc0, wb0, sem0)
        process(rc0, wb0)

        @pl.when(g0 + 2 < NCH)
        def _():
            start(g0 + 2, rc0, wb0, sem0)

        wait(rc1, wb1, sem1)
        process(rc1, wb1)
        return 0

    lax.fori_loop(0, NCH // 2, outer, 0)
    pltpu.sync_copy(acc, p_hbm.at[cid, pl.ds(sid * DPT, DPT)])


def _sc_agg64(xwp, ei, ew):
    k = pl.kernel(
        _sc_agg64_body,
        out_type=jax.ShapeDtypeStruct((NC, D_HID, NPAD), jnp.float32),
        mesh=_mesh(),
        compiler_params=pltpu.CompilerParams(needs_layout_passes=False),
        scratch_types=[
            pltpu.VMEM((NPK, NPAD), jnp.float32),
            pltpu.VMEM((DPT, NPAD), jnp.float32),
            pltpu.VMEM((2, CHUNK), jnp.int32),
            pltpu.VMEM((2, CHUNK), jnp.int32),
            pltpu.VMEM((CHUNK,), jnp.float32),
            pltpu.VMEM((CHUNK,), jnp.float32),
            pltpu.SemaphoreType.DMA,
            pltpu.SemaphoreType.DMA,
        ],
    )
    return k(xwp, ei, ew)


def _tc2_body(p_ref, xwT_ref, dis_ref, b1_ref, w2_ref, hw2_ref):
    dis = dis_ref[...]
    s = (p_ref[0] + p_ref[1] + xwT_ref[...]) * dis[None, :] + b1_ref[...]
    h = jnp.where(s > 0, s, jnp.exp(jnp.minimum(s, 0.0)) - 1.0)
    hw2_ref[...] = jnp.sum(h * w2_ref[...], axis=0) * dis


def _tc2(p, xwT, dis, b1c, W2):
    B = 2048
    grid = NPAD // B
    return pl.pallas_call(
        _tc2_body,
        grid=(grid,),
        in_specs=[
            pl.BlockSpec((NC, D_HID, B), lambda j: (0, 0, j)),
            pl.BlockSpec((D_HID, B), lambda j: (0, j)),
            pl.BlockSpec((B,), lambda j: (j,)),
            pl.BlockSpec((D_HID, 1), lambda j: (0, 0)),
            pl.BlockSpec((D_HID, 1), lambda j: (0, 0)),
        ],
        out_specs=pl.BlockSpec((B,), lambda j: (j,)),
        out_shape=jax.ShapeDtypeStruct((NPAD,), jnp.float32),
    )(p, xwT, dis, b1c, W2)


def _sc_agg1_body(hw2_hbm, row_hbm, col_hbm, ew_hbm, q_hbm,
                  hv, acc, rbuf, cbuf, wbuf):
    cid = lax.axis_index("c")
    sid = lax.axis_index("s")
    wid = cid * NS + sid
    pltpu.sync_copy(hw2_hbm, hv)
    pltpu.sync_copy(row_hbm.at[pl.ds(wid * EPT, EPT)], rbuf)
    pltpu.sync_copy(col_hbm.at[pl.ds(wid * EPT, EPT)], cbuf)
    pltpu.sync_copy(ew_hbm.at[pl.ds(wid * EPT, EPT)], wbuf)
    _zero_vmem(acc, NPAD)

    @plsc.parallel_loop(0, EPT, step=L, unroll=4)
    def body(i):
        r = rbuf[pl.ds(i, L)]
        c = cbuf[pl.ds(i, L)]
        w = wbuf[pl.ds(i, L)]
        hr = plsc.load_gather(hv, [r])
        plsc.addupdate_scatter(acc, [c], hr * w)

    pltpu.sync_copy(acc, q_hbm.at[wid])


def _sc_agg1(hw2, row, col, ew):
    k = pl.kernel(
        _sc_agg1_body,
        out_type=jax.ShapeDtypeStruct((NW, NPAD), jnp.float32),
        mesh=_mesh(),
        compiler_params=pltpu.CompilerParams(needs_layout_passes=False),
        scratch_types=[
            pltpu.VMEM((NPAD,), jnp.float32),
            pltpu.VMEM((NPAD,), jnp.float32),
            pltpu.VMEM((EPT,), jnp.int32),
            pltpu.VMEM((EPT,), jnp.int32),
            pltpu.VMEM((EPT,), jnp.float32),
        ],
    )
    return k(hw2, row, col, ew)


def _tc3_body(q_ref, hw2_ref, dis_ref, b2_ref, out_ref):
    z = (jnp.sum(q_ref[...], axis=0) + hw2_ref[...]) * dis_ref[...] + b2_ref[0, 0]
    out_ref[...] = 1.0 / (1.0 + jnp.exp(-z))


def _tc3(q, hw2, dis, b2c):
    B = 2048
    grid = NPAD // B
    return pl.pallas_call(
        _tc3_body,
        grid=(grid,),
        in_specs=[
            pl.BlockSpec((NW, B), lambda j: (0, j)),
            pl.BlockSpec((B,), lambda j: (j,)),
            pl.BlockSpec((B,), lambda j: (j,)),
            pl.BlockSpec((1, 1), lambda j: (0, 0)),
        ],
        out_specs=pl.BlockSpec((B,), lambda j: (j,)),
        out_shape=jax.ShapeDtypeStruct((NPAD,), jnp.float32),
    )(q, hw2, dis, b2c)


@jax.jit
def kernel(x, edge_index, edge_weight, W1, b1, W2, b2):
    row = edge_index[0]
    col = edge_index[1]
    xp = jnp.zeros((NPAD, D_IN), jnp.float32).at[:N].set(x)
    b1c = b1.reshape(D_HID, 1)
    b2c = b2.reshape(1, 1)

    degp = _sc_deg(col, edge_weight)
    xwT, xwp, dis = _tc1(xp, W1, degp)
    p = _sc_agg64(xwp, edge_index, edge_weight)
    hw2 = _tc2(p, xwT, dis, b1c, W2)
    q = _sc_agg1(hw2, row, col, edge_weight)
    out = _tc3(q, hw2, dis, b2c)
    return out[:N].reshape(N, 1)


# speedup vs baseline: 1.0494x; 1.0005x over previous
import jax
import jax.numpy as jnp
from jax import lax
from jax.experimental import pallas as pl
from jax.experimental.pallas import tpu as pltpu
from jax.experimental.pallas import tpu_sc as plsc

N = 10000
E = 320000
D_IN = 128
D_HID = 64

NC = 2
NS = 16
NW = NC * NS
L = 16

NPAD = 10240
DPT = D_HID // NS
NPK = DPT // 2
EPT = E // NW
EHALF = E // NC
CHUNK = 3200

_mesh = lambda: plsc.VectorSubcoreMesh(core_axis_name="c", subcore_axis_name="s")


def _zero_vmem(ref, total):
    z = jnp.zeros((L,), jnp.float32)

    def body(i, _):
        ref[pl.ds(i * L, L)] = z
        return 0

    lax.fori_loop(0, total // L, body, 0)


def _zero_vmem2(ref, rows, cols):
    z = jnp.zeros((L,), jnp.float32)

    def body(i, _):
        for j in range(rows):
            ref[j, pl.ds(i * L, L)] = z
        return 0

    lax.fori_loop(0, cols // L, body, 0)


def _sc_deg_body(col_hbm, ew_hbm, degp_hbm, cbuf, wbuf, acc):
    cid = lax.axis_index("c")
    sid = lax.axis_index("s")
    wid = cid * NS + sid
    pltpu.sync_copy(col_hbm.at[pl.ds(wid * EPT, EPT)], cbuf)
    pltpu.sync_copy(ew_hbm.at[pl.ds(wid * EPT, EPT)], wbuf)
    _zero_vmem(acc, NPAD)

    @plsc.parallel_loop(0, EPT, step=L, unroll=4)
    def body(i):
        c = cbuf[pl.ds(i, L)]
        w = wbuf[pl.ds(i, L)]
        plsc.addupdate_scatter(acc, [c], w)

    pltpu.sync_copy(acc, degp_hbm.at[wid])


def _sc_deg(col, ew):
    k = pl.kernel(
        _sc_deg_body,
        out_type=jax.ShapeDtypeStruct((NW, NPAD), jnp.float32),
        mesh=_mesh(),
        compiler_params=pltpu.CompilerParams(needs_layout_passes=False, skip_device_barrier=True),
        scratch_types=[
            pltpu.VMEM((EPT,), jnp.int32),
            pltpu.VMEM((EPT,), jnp.float32),
            pltpu.VMEM((NPAD,), jnp.float32),
        ],
    )
    return k(col, ew)


def _tc1_body(x_ref, w1_ref, degp_ref, xwT_ref, xwp_ref, dis_ref):
    xb = x_ref[...]
    w = w1_ref[...]
    mm = lax.dot_general(
        w, xb, (((0,), (1,)), ((), ())), preferred_element_type=jnp.float32
    )
    deg = jnp.sum(degp_ref[...], axis=0) + 1.0
    dis = lax.rsqrt(deg)
    dis_ref[...] = dis
    xwT = mm * dis[None, :]
    xwT_ref[...] = xwT
    bf = lax.convert_element_type(xwT, jnp.bfloat16)
    bits = lax.convert_element_type(
        lax.bitcast_convert_type(bf, jnp.uint16), jnp.uint32
    )
    pairs = bits.reshape(D_HID // 2, 2, bits.shape[-1])
    packed = (pairs[:, 1, :] << 16) | pairs[:, 0, :]
    xwp_ref[...] = lax.bitcast_convert_type(packed, jnp.float32)


def _tc1(xp, W1, degp):
    B = 2048
    grid = NPAD // B
    return pl.pallas_call(
        _tc1_body,
        grid=(grid,),
        in_specs=[
            pl.BlockSpec((B, D_IN), lambda j: (j, 0)),
            pl.BlockSpec((D_IN, D_HID), lambda j: (0, 0)),
            pl.BlockSpec((NW, B), lambda j: (0, j)),
        ],
        out_specs=[
            pl.BlockSpec((D_HID, B), lambda j: (0, j)),
            pl.BlockSpec((D_HID // 2, B), lambda j: (0, j)),
            pl.BlockSpec((B,), lambda j: (j,)),
        ],
        out_shape=[
            jax.ShapeDtypeStruct((D_HID, NPAD), jnp.float32),
            jax.ShapeDtypeStruct((D_HID // 2, NPAD), jnp.float32),
            jax.ShapeDtypeStruct((NPAD,), jnp.float32),
        ],
    )(xp, W1, degp)


def _sc_agg64_body(xwp_hbm, ei_hbm, ew_hbm, p_hbm,
                   xwc, acc, rc0, rc1, wb0, wb1, sem0, sem1):
    cid = lax.axis_index("c")
    sid = lax.axis_index("s")
    base = cid * EHALF
    NCH = EHALF // CHUNK
    kvs = [jnp.full((L,), k, jnp.int32) for k in range(NPK)]
    jvs = [jnp.full((L,), j, jnp.int32) for j in range(DPT)]

    def start(g, rc, wb, sem):
        off = base + g * CHUNK
        pltpu.async_copy(ei_hbm.at[:, pl.ds(off, CHUNK)], rc, sem)
        pltpu.async_copy(ew_hbm.at[pl.ds(off, CHUNK)], wb, sem)

    def wait(rc, wb, sem):
        pltpu.make_async_copy(ei_hbm.at[:, pl.ds(0, CHUNK)], rc, sem).wait()
        pltpu.make_async_copy(ew_hbm.at[pl.ds(0, CHUNK)], wb, sem).wait()

    def process(rc, wb):
        @plsc.parallel_loop(0, CHUNK, step=L, unroll=4)
        def body(i):
            r = rc[0, pl.ds(i, L)]
            c = rc[1, pl.ds(i, L)]
            w = wb[pl.ds(i, L)]
            for k in range(NPK):
                g = plsc.load_gather(xwc, [kvs[k], r])
                ab = plsc.bitcast(g, jnp.bfloat16)
                va, vb = plsc.unpack(
                    ab, format=plsc.PackFormat.INTERLEAVED,
                    preferred_element_type=jnp.float32,
                )
                plsc.addupdate_scatter(acc, [jvs[2 * k], c], va * w)
                plsc.addupdate_scatter(acc, [jvs[2 * k + 1], c], vb * w)

    start(0, rc0, wb0, sem0)
    pltpu.sync_copy(xwp_hbm.at[pl.ds(sid * NPK, NPK)], xwc)
    _zero_vmem2(acc, DPT, NPAD)

    def outer(gg, _):
        g0 = gg * 2
        start(g0 + 1, rc1, wb1, sem1)
        wait(rc0, wb0, sem0)
        process(rc0, wb0)

        @pl.when(g0 + 2 < NCH)
        def _():
            start(g0 + 2, rc0, wb0, sem0)

        wait(rc1, wb1, sem1)
        process(rc1, wb1)
        return 0

    lax.fori_loop(0, NCH // 2, outer, 0)
    pltpu.sync_copy(acc, p_hbm.at[cid, pl.ds(sid * DPT, DPT)])


def _sc_agg64(xwp, ei, ew):
    k = pl.kernel(
        _sc_agg64_body,
        out_type=jax.ShapeDtypeStruct((NC, D_HID, NPAD), jnp.float32),
        mesh=_mesh(),
        compiler_params=pltpu.CompilerParams(needs_layout_passes=False, skip_device_barrier=True),
        scratch_types=[
            pltpu.VMEM((NPK, NPAD), jnp.float32),
            pltpu.VMEM((DPT, NPAD), jnp.float32),
            pltpu.VMEM((2, CHUNK), jnp.int32),
            pltpu.VMEM((2, CHUNK), jnp.int32),
            pltpu.VMEM((CHUNK,), jnp.float32),
            pltpu.VMEM((CHUNK,), jnp.float32),
            pltpu.SemaphoreType.DMA,
            pltpu.SemaphoreType.DMA,
        ],
    )
    return k(xwp, ei, ew)


def _tc2_body(p_ref, xwT_ref, dis_ref, b1_ref, w2_ref, hw2_ref):
    dis = dis_ref[...]
    s = (p_ref[0] + p_ref[1] + xwT_ref[...]) * dis[None, :] + b1_ref[...]
    h = jnp.where(s > 0, s, jnp.exp(jnp.minimum(s, 0.0)) - 1.0)
    hw2_ref[...] = jnp.sum(h * w2_ref[...], axis=0) * dis


def _tc2(p, xwT, dis, b1c, W2):
    B = 2048
    grid = NPAD // B
    return pl.pallas_call(
        _tc2_body,
        grid=(grid,),
        in_specs=[
            pl.BlockSpec((NC, D_HID, B), lambda j: (0, 0, j)),
            pl.BlockSpec((D_HID, B), lambda j: (0, j)),
            pl.BlockSpec((B,), lambda j: (j,)),
            pl.BlockSpec((D_HID, 1), lambda j: (0, 0)),
            pl.BlockSpec((D_HID, 1), lambda j: (0, 0)),
        ],
        out_specs=pl.BlockSpec((B,), lambda j: (j,)),
        out_shape=jax.ShapeDtypeStruct((NPAD,), jnp.float32),
    )(p, xwT, dis, b1c, W2)


def _sc_agg1_body(hw2_hbm, row_hbm, col_hbm, ew_hbm, q_hbm,
                  hv, acc, rbuf, cbuf, wbuf):
    cid = lax.axis_index("c")
    sid = lax.axis_index("s")
    wid = cid * NS + sid
    pltpu.sync_copy(hw2_hbm, hv)
    pltpu.sync_copy(row_hbm.at[pl.ds(wid * EPT, EPT)], rbuf)
    pltpu.sync_copy(col_hbm.at[pl.ds(wid * EPT, EPT)], cbuf)
    pltpu.sync_copy(ew_hbm.at[pl.ds(wid * EPT, EPT)], wbuf)
    _zero_vmem(acc, NPAD)

    @plsc.parallel_loop(0, EPT, step=L, unroll=4)
    def body(i):
        r = rbuf[pl.ds(i, L)]
        c = cbuf[pl.ds(i, L)]
        w = wbuf[pl.ds(i, L)]
        hr = plsc.load_gather(hv, [r])
        plsc.addupdate_scatter(acc, [c], hr * w)

    pltpu.sync_copy(acc, q_hbm.at[wid])


def _sc_agg1(hw2, row, col, ew):
    k = pl.kernel(
        _sc_agg1_body,
        out_type=jax.ShapeDtypeStruct((NW, NPAD), jnp.float32),
        mesh=_mesh(),
        compiler_params=pltpu.CompilerParams(needs_layout_passes=False, skip_device_barrier=True),
        scratch_types=[
            pltpu.VMEM((NPAD,), jnp.float32),
            pltpu.VMEM((NPAD,), jnp.float32),
            pltpu.VMEM((EPT,), jnp.int32),
            pltpu.VMEM((EPT,), jnp.int32),
            pltpu.VMEM((EPT,), jnp.float32),
        ],
    )
    return k(hw2, row, col, ew)


def _tc3_body(q_ref, hw2_ref, dis_ref, b2_ref, out_ref):
    z = (jnp.sum(q_ref[...], axis=0) + hw2_ref[...]) * dis_ref[...] + b2_ref[0, 0]
    out_ref[...] = 1.0 / (1.0 + jnp.exp(-z))


def _tc3(q, hw2, dis, b2c):
    B = 2048
    grid = NPAD // B
    return pl.pallas_call(
        _tc3_body,
        grid=(grid,),
        in_specs=[
            pl.BlockSpec((NW, B), lambda j: (0, j)),
            pl.BlockSpec((B,), lambda j: (j,)),
            pl.BlockSpec((B,), lambda j: (j,)),
            pl.BlockSpec((1, 1), lambda j: (0, 0)),
        ],
        out_specs=pl.BlockSpec((B,), lambda j: (j,)),
        out_shape=jax.ShapeDtypeStruct((NPAD,), jnp.float32),
    )(q, hw2, dis, b2c)


@jax.jit
def kernel(x, edge_index, edge_weight, W1, b1, W2, b2):
    row = edge_index[0]
    col = edge_index[1]
    xp = jnp.zeros((NPAD, D_IN), jnp.float32).at[:N].set(x)
    b1c = b1.reshape(D_HID, 1)
    b2c = b2.reshape(1, 1)

    degp = _sc_deg(col, edge_weight)
    xwT, xwp, dis = _tc1(xp, W1, degp)
    p = _sc_agg64(xwp, edge_index, edge_weight)
    hw2 = _tc2(p, xwT, dis, b1c, W2)
    q = _sc_agg1(hw2, row, col, edge_weight)
    out = _tc3(q, hw2, dis, b2c)
    return out[:N].reshape(N, 1)


# speedup vs baseline: 1.0951x; 1.0436x over previous
import jax
import jax.numpy as jnp
from jax import lax
from jax.experimental import pallas as pl
from jax.experimental.pallas import tpu as pltpu
from jax.experimental.pallas import tpu_sc as plsc

N = 10000
E = 320000
D_IN = 128
D_HID = 64

NC = 2
NS = 16
NW = NC * NS
L = 16

NPAD = 10240
DPT = D_HID // NS
NPK = DPT // 2
EPT = E // NW
EHALF = E // NC
CHUNK = 3200

_mesh = lambda: plsc.VectorSubcoreMesh(core_axis_name="c", subcore_axis_name="s")


def _zero_vmem(ref, total):
    z = jnp.zeros((L,), jnp.float32)

    def body(i, _):
        ref[pl.ds(i * L, L)] = z
        return 0

    lax.fori_loop(0, total // L, body, 0)


def _zero_vmem2(ref, rows, cols):
    z = jnp.zeros((L,), jnp.float32)

    def body(i, _):
        for j in range(rows):
            ref[j, pl.ds(i * L, L)] = z
        return 0

    lax.fori_loop(0, cols // L, body, 0)


def _sc_deg_body(col_hbm, ew_hbm, degp_hbm, cbuf, wbuf, acc, sem):
    cid = lax.axis_index("c")
    sid = lax.axis_index("s")
    wid = cid * NS + sid
    pltpu.async_copy(col_hbm.at[pl.ds(wid * EPT, EPT)], cbuf, sem)
    pltpu.async_copy(ew_hbm.at[pl.ds(wid * EPT, EPT)], wbuf, sem)
    _zero_vmem(acc, NPAD)
    pltpu.make_async_copy(col_hbm.at[pl.ds(0, EPT)], cbuf, sem).wait()
    pltpu.make_async_copy(ew_hbm.at[pl.ds(0, EPT)], wbuf, sem).wait()

    @plsc.parallel_loop(0, EPT, step=L, unroll=4)
    def body(i):
        c = cbuf[pl.ds(i, L)]
        w = wbuf[pl.ds(i, L)]
        plsc.addupdate_scatter(acc, [c], w)

    pltpu.sync_copy(acc, degp_hbm.at[wid])


def _sc_deg(col, ew):
    k = pl.kernel(
        _sc_deg_body,
        out_type=jax.ShapeDtypeStruct((NW, NPAD), jnp.float32),
        mesh=_mesh(),
        compiler_params=pltpu.CompilerParams(needs_layout_passes=False),
        scratch_types=[
            pltpu.VMEM((EPT,), jnp.int32),
            pltpu.VMEM((EPT,), jnp.float32),
            pltpu.VMEM((NPAD,), jnp.float32),
            pltpu.SemaphoreType.DMA,
        ],
    )
    return k(col, ew)


def _tc1_body(x_ref, w1_ref, degp_ref, xwT_ref, xwp_ref, dis_ref):
    xb = x_ref[...]
    w = w1_ref[...]
    mm = lax.dot_general(
        w, xb, (((0,), (1,)), ((), ())), preferred_element_type=jnp.float32
    )
    deg = jnp.sum(degp_ref[...], axis=0) + 1.0
    dis = lax.rsqrt(deg)
    dis_ref[...] = dis
    xwT = mm * dis[None, :]
    xwT_ref[...] = xwT
    bf = lax.convert_element_type(xwT, jnp.bfloat16)
    bits = lax.convert_element_type(
        lax.bitcast_convert_type(bf, jnp.uint16), jnp.uint32
    )
    pairs = bits.reshape(D_HID // 2, 2, bits.shape[-1])
    packed = (pairs[:, 1, :] << 16) | pairs[:, 0, :]
    xwp_ref[...] = lax.bitcast_convert_type(packed, jnp.float32)


def _tc1(xp, W1, degp):
    B = 2048
    grid = NPAD // B
    return pl.pallas_call(
        _tc1_body,
        grid=(grid,),
        in_specs=[
            pl.BlockSpec((B, D_IN), lambda j: (j, 0)),
            pl.BlockSpec((D_IN, D_HID), lambda j: (0, 0)),
            pl.BlockSpec((NW, B), lambda j: (0, j)),
        ],
        out_specs=[
            pl.BlockSpec((D_HID, B), lambda j: (0, j)),
            pl.BlockSpec((D_HID // 2, B), lambda j: (0, j)),
            pl.BlockSpec((B,), lambda j: (j,)),
        ],
        out_shape=[
            jax.ShapeDtypeStruct((D_HID, NPAD), jnp.float32),
            jax.ShapeDtypeStruct((D_HID // 2, NPAD), jnp.float32),
            jax.ShapeDtypeStruct((NPAD,), jnp.float32),
        ],
    )(xp, W1, degp)


def _sc_agg64_body(xwp_hbm, ei_hbm, ew_hbm, p_hbm,
                   xwc, acc, rc0, rc1, wb0, wb1, sem0, sem1):
    cid = lax.axis_index("c")
    sid = lax.axis_index("s")
    base = cid * EHALF
    NCH = EHALF // CHUNK
    kvs = [jnp.full((L,), k, jnp.int32) for k in range(NPK)]
    jvs = [jnp.full((L,), j, jnp.int32) for j in range(DPT)]

    def start(g, rc, wb, sem):
        off = base + g * CHUNK
        pltpu.async_copy(ei_hbm.at[:, pl.ds(off, CHUNK)], rc, sem)
        pltpu.async_copy(ew_hbm.at[pl.ds(off, CHUNK)], wb, sem)

    def wait(rc, wb, sem):
        pltpu.make_async_copy(ei_hbm.at[:, pl.ds(0, CHUNK)], rc, sem).wait()
        pltpu.make_async_copy(ew_hbm.at[pl.ds(0, CHUNK)], wb, sem).wait()

    def process(rc, wb):
        @plsc.parallel_loop(0, CHUNK, step=L, unroll=4)
        def body(i):
            r = rc[0, pl.ds(i, L)]
            c = rc[1, pl.ds(i, L)]
            w = wb[pl.ds(i, L)]
            for k in range(NPK):
                g = plsc.load_gather(xwc, [kvs[k], r])
                ab = plsc.bitcast(g, jnp.bfloat16)
                va, vb = plsc.unpack(
                    ab, format=plsc.PackFormat.INTERLEAVED,
                    preferred_element_type=jnp.float32,
                )
                plsc.addupdate_scatter(acc, [jvs[2 * k], c], va * w)
                plsc.addupdate_scatter(acc, [jvs[2 * k + 1], c], vb * w)

    start(0, rc0, wb0, sem0)
    pltpu.sync_copy(xwp_hbm.at[pl.ds(sid * NPK, NPK)], xwc)
    _zero_vmem2(acc, DPT, NPAD)

    def outer(gg, _):
        g0 = gg * 2
        start(g0 + 1, rc1, wb1, sem1)
        wait(rc0, wb0, sem0)
        process(rc0, wb0)

        @pl.when(g0 + 2 < NCH)
        def _():
            start(g0 + 2, rc0, wb0, sem0)

        wait(rc1, wb1, sem1)
        process(rc1, wb1)
        return 0

    lax.fori_loop(0, NCH // 2, outer, 0)
    pltpu.sync_copy(acc, p_hbm.at[cid, pl.ds(sid * DPT, DPT)])


def _sc_agg64(xwp, ei, ew):
    k = pl.kernel(
        _sc_agg64_body,
        out_type=jax.ShapeDtypeStruct((NC, D_HID, NPAD), jnp.float32),
        mesh=_mesh(),
        compiler_params=pltpu.CompilerParams(needs_layout_passes=False),
        scratch_types=[
            pltpu.VMEM((NPK, NPAD), jnp.float32),
            pltpu.VMEM((DPT, NPAD), jnp.float32),
            pltpu.VMEM((2, CHUNK), jnp.int32),
            pltpu.VMEM((2, CHUNK), jnp.int32),
            pltpu.VMEM((CHUNK,), jnp.float32),
            pltpu.VMEM((CHUNK,), jnp.float32),
            pltpu.SemaphoreType.DMA,
            pltpu.SemaphoreType.DMA,
        ],
    )
    return k(xwp, ei, ew)


def _tc2_body(p_ref, xwT_ref, dis_ref, b1_ref, w2_ref, hw2_ref):
    dis = dis_ref[...]
    s = (p_ref[0] + p_ref[1] + xwT_ref[...]) * dis[None, :] + b1_ref[...]
    h = jnp.where(s > 0, s, jnp.exp(jnp.minimum(s, 0.0)) - 1.0)
    hw2_ref[...] = jnp.sum(h * w2_ref[...], axis=0) * dis


def _tc2(p, xwT, dis, b1c, W2):
    B = 2048
    grid = NPAD // B
    return pl.pallas_call(
        _tc2_body,
        grid=(grid,),
        in_specs=[
            pl.BlockSpec((NC, D_HID, B), lambda j: (0, 0, j)),
            pl.BlockSpec((D_HID, B), lambda j: (0, j)),
            pl.BlockSpec((B,), lambda j: (j,)),
            pl.BlockSpec((D_HID, 1), lambda j: (0, 0)),
            pl.BlockSpec((D_HID, 1), lambda j: (0, 0)),
        ],
        out_specs=pl.BlockSpec((B,), lambda j: (j,)),
        out_shape=jax.ShapeDtypeStruct((NPAD,), jnp.float32),
    )(p, xwT, dis, b1c, W2)


def _sc_agg1_body(hw2_hbm, row_hbm, col_hbm, ew_hbm, q_hbm,
                  hv, acc, rbuf, cbuf, wbuf, sem):
    cid = lax.axis_index("c")
    sid = lax.axis_index("s")
    wid = cid * NS + sid
    pltpu.async_copy(hw2_hbm, hv, sem)
    pltpu.async_copy(row_hbm.at[pl.ds(wid * EPT, EPT)], rbuf, sem)
    pltpu.async_copy(col_hbm.at[pl.ds(wid * EPT, EPT)], cbuf, sem)
    pltpu.async_copy(ew_hbm.at[pl.ds(wid * EPT, EPT)], wbuf, sem)
    _zero_vmem(acc, NPAD)
    pltpu.make_async_copy(hw2_hbm, hv, sem).wait()
    pltpu.make_async_copy(row_hbm.at[pl.ds(0, EPT)], rbuf, sem).wait()
    pltpu.make_async_copy(col_hbm.at[pl.ds(0, EPT)], cbuf, sem).wait()
    pltpu.make_async_copy(ew_hbm.at[pl.ds(0, EPT)], wbuf, sem).wait()

    @plsc.parallel_loop(0, EPT, step=L, unroll=4)
    def body(i):
        r = rbuf[pl.ds(i, L)]
        c = cbuf[pl.ds(i, L)]
        w = wbuf[pl.ds(i, L)]
        hr = plsc.load_gather(hv, [r])
        plsc.addupdate_scatter(acc, [c], hr * w)

    pltpu.sync_copy(acc, q_hbm.at[wid])


def _sc_agg1(hw2, row, col, ew):
    k = pl.kernel(
        _sc_agg1_body,
        out_type=jax.ShapeDtypeStruct((NW, NPAD), jnp.float32),
        mesh=_mesh(),
        compiler_params=pltpu.CompilerParams(needs_layout_passes=False),
        scratch_types=[
            pltpu.VMEM((NPAD,), jnp.float32),
            pltpu.VMEM((NPAD,), jnp.float32),
            pltpu.VMEM((EPT,), jnp.int32),
            pltpu.VMEM((EPT,), jnp.int32),
            pltpu.VMEM((EPT,), jnp.float32),
            pltpu.SemaphoreType.DMA,
        ],
    )
    return k(hw2, row, col, ew)


def _tc3_body(q_ref, hw2_ref, dis_ref, b2_ref, out_ref):
    z = (jnp.sum(q_ref[...], axis=0) + hw2_ref[...]) * dis_ref[...] + b2_ref[0, 0]
    out_ref[...] = 1.0 / (1.0 + jnp.exp(-z))


def _tc3(q, hw2, dis, b2c):
    B = 2048
    grid = NPAD // B
    return pl.pallas_call(
        _tc3_body,
        grid=(grid,),
        in_specs=[
            pl.BlockSpec((NW, B), lambda j: (0, j)),
            pl.BlockSpec((B,), lambda j: (j,)),
            pl.BlockSpec((B,), lambda j: (j,)),
            pl.BlockSpec((1, 1), lambda j: (0, 0)),
        ],
        out_specs=pl.BlockSpec((B,), lambda j: (j,)),
        out_shape=jax.ShapeDtypeStruct((NPAD,), jnp.float32),
    )(q, hw2, dis, b2c)


@jax.jit
def kernel(x, edge_index, edge_weight, W1, b1, W2, b2):
    row = edge_index[0]
    col = edge_index[1]
    xp = jnp.zeros((NPAD, D_IN), jnp.float32).at[:N].set(x)
    b1c = b1.reshape(D_HID, 1)
    b2c = b2.reshape(1, 1)

    degp = _sc_deg(col, edge_weight)
    xwT, xwp, dis = _tc1(xp, W1, degp)
    p = _sc_agg64(xwp, edge_index, edge_weight)
    hw2 = _tc2(p, xwT, dis, b1c, W2)
    q = _sc_agg1(hw2, row, col, edge_weight)
    out = _tc3(q, hw2, dis, b2c)
    return out[:N].reshape(N, 1)


# speedup vs baseline: 1.1588x; 1.0581x over previous
import jax
import jax.numpy as jnp
from jax import lax
from jax.experimental import pallas as pl
from jax.experimental.pallas import tpu as pltpu
from jax.experimental.pallas import tpu_sc as plsc

N = 10000
E = 320000
D_IN = 128
D_HID = 64

NC = 2
NS = 16
NW = NC * NS
L = 16

NPAD = 10240
DPT = D_HID // NS
NPK = DPT // 2
EPT = E // NW
EHALF = E // NC
CHUNK = 3200

_mesh = lambda: plsc.VectorSubcoreMesh(core_axis_name="c", subcore_axis_name="s")


def _zero_vmem(ref, total):
    z = jnp.zeros((L,), jnp.float32)

    def body(i, _):
        ref[pl.ds(i * L, L)] = z
        return 0

    lax.fori_loop(0, total // L, body, 0)


def _zero_vmem2(ref, rows, cols):
    z = jnp.zeros((L,), jnp.float32)

    def body(i, _):
        for j in range(rows):
            ref[j, pl.ds(i * L, L)] = z
        return 0

    lax.fori_loop(0, cols // L, body, 0)


def _sc_deg_body(col_hbm, ew_hbm, degp_hbm, cbuf, wbuf, acc, sem):
    cid = lax.axis_index("c")
    sid = lax.axis_index("s")
    wid = cid * NS + sid
    pltpu.async_copy(col_hbm.at[pl.ds(wid * EPT, EPT)], cbuf, sem)
    pltpu.async_copy(ew_hbm.at[pl.ds(wid * EPT, EPT)], wbuf, sem)
    _zero_vmem(acc, NPAD)
    pltpu.make_async_copy(col_hbm.at[pl.ds(0, EPT)], cbuf, sem).wait()
    pltpu.make_async_copy(ew_hbm.at[pl.ds(0, EPT)], wbuf, sem).wait()

    @plsc.parallel_loop(0, EPT, step=L, unroll=4)
    def body(i):
        c = cbuf[pl.ds(i, L)]
        w = wbuf[pl.ds(i, L)]
        plsc.addupdate_scatter(acc, [c], w)

    pltpu.sync_copy(acc, degp_hbm.at[wid])


def _sc_deg(col, ew):
    k = pl.kernel(
        _sc_deg_body,
        out_type=jax.ShapeDtypeStruct((NW, NPAD), jnp.float32),
        mesh=_mesh(),
        compiler_params=pltpu.CompilerParams(needs_layout_passes=False),
        scratch_types=[
            pltpu.VMEM((EPT,), jnp.int32),
            pltpu.VMEM((EPT,), jnp.float32),
            pltpu.VMEM((NPAD,), jnp.float32),
            pltpu.SemaphoreType.DMA,
        ],
    )
    return k(col, ew)


def _tc1_body(x_ref, w1_ref, degp_ref, xwT_ref, xwp_ref, dis_ref):
    xb = x_ref[...]
    w = w1_ref[...]
    mm = lax.dot_general(
        w, xb, (((0,), (1,)), ((), ())), preferred_element_type=jnp.float32
    )
    deg = jnp.sum(degp_ref[...], axis=0) + 1.0
    dis = lax.rsqrt(deg)
    dis_ref[...] = dis
    xwT = mm * dis[None, :]
    xwT_ref[...] = xwT
    bf = lax.convert_element_type(xwT, jnp.bfloat16)
    bits = lax.convert_element_type(
        lax.bitcast_convert_type(bf, jnp.uint16), jnp.uint32
    )
    pairs = bits.reshape(D_HID // 2, 2, bits.shape[-1])
    packed = (pairs[:, 1, :] << 16) | pairs[:, 0, :]
    xwp_ref[...] = lax.bitcast_convert_type(packed, jnp.float32)


def _tc1(xp, W1, degp):
    B = 2048
    grid = NPAD // B
    return pl.pallas_call(
        _tc1_body,
        grid=(grid,),
        in_specs=[
            pl.BlockSpec((B, D_IN), lambda j: (j, 0)),
            pl.BlockSpec((D_IN, D_HID), lambda j: (0, 0)),
            pl.BlockSpec((NW, B), lambda j: (0, j)),
        ],
        out_specs=[
            pl.BlockSpec((D_HID, B), lambda j: (0, j)),
            pl.BlockSpec((D_HID // 2, B), lambda j: (0, j)),
            pl.BlockSpec((B,), lambda j: (j,)),
        ],
        out_shape=[
            jax.ShapeDtypeStruct((D_HID, NPAD), jnp.float32),
            jax.ShapeDtypeStruct((D_HID // 2, NPAD), jnp.float32),
            jax.ShapeDtypeStruct((NPAD,), jnp.float32),
        ],
    )(xp, W1, degp)


def _sc_agg64_body(xwp_hbm, rc_hbm, ew_hbm, p_hbm,
                   xwc, acc, rc0, rc1, wb0, wb1, sem0, sem1):
    cid = lax.axis_index("c")
    sid = lax.axis_index("s")
    base = cid * EHALF
    NCH = EHALF // CHUNK
    kvs = [jnp.full((L,), k, jnp.int32) for k in range(NPK)]
    jvs = [jnp.full((L,), j, jnp.int32) for j in range(DPT)]

    def start(g, rc, wb, sem):
        off = base + g * CHUNK
        pltpu.async_copy(rc_hbm.at[pl.ds(off, CHUNK)], rc, sem)
        pltpu.async_copy(ew_hbm.at[pl.ds(off, CHUNK)], wb, sem)

    def wait(rc, wb, sem):
        pltpu.make_async_copy(rc_hbm.at[pl.ds(0, CHUNK)], rc, sem).wait()
        pltpu.make_async_copy(ew_hbm.at[pl.ds(0, CHUNK)], wb, sem).wait()

    def process(rc, wb):
        @plsc.parallel_loop(0, CHUNK, step=L, unroll=4)
        def body(i):
            v = rc[pl.ds(i, L)]
            r = v & jnp.int32(0xFFFF)
            c = jnp.right_shift(v, 16)
            w = wb[pl.ds(i, L)]
            for k in range(NPK):
                g = plsc.load_gather(xwc, [kvs[k], r])
                ab = plsc.bitcast(g, jnp.bfloat16)
                va, vb = plsc.unpack(
                    ab, format=plsc.PackFormat.INTERLEAVED,
                    preferred_element_type=jnp.float32,
                )
                plsc.addupdate_scatter(acc, [jvs[2 * k], c], va * w)
                plsc.addupdate_scatter(acc, [jvs[2 * k + 1], c], vb * w)

    start(0, rc0, wb0, sem0)
    pltpu.sync_copy(xwp_hbm.at[pl.ds(sid * NPK, NPK)], xwc)
    _zero_vmem2(acc, DPT, NPAD)

    def outer(gg, _):
        g0 = gg * 2
        start(g0 + 1, rc1, wb1, sem1)
        wait(rc0, wb0, sem0)
        process(rc0, wb0)

        @pl.when(g0 + 2 < NCH)
        def _():
            start(g0 + 2, rc0, wb0, sem0)

        wait(rc1, wb1, sem1)
        process(rc1, wb1)
        return 0

    lax.fori_loop(0, NCH // 2, outer, 0)
    pltpu.sync_copy(acc, p_hbm.at[cid, pl.ds(sid * DPT, DPT)])


def _sc_agg64(xwp, rc, ew):
    k = pl.kernel(
        _sc_agg64_body,
        out_type=jax.ShapeDtypeStruct((NC, D_HID, NPAD), jnp.float32),
        mesh=_mesh(),
        compiler_params=pltpu.CompilerParams(needs_layout_passes=False),
        scratch_types=[
            pltpu.VMEM((NPK, NPAD), jnp.float32),
            pltpu.VMEM((DPT, NPAD), jnp.float32),
            pltpu.VMEM((CHUNK,), jnp.int32),
            pltpu.VMEM((CHUNK,), jnp.int32),
            pltpu.VMEM((CHUNK,), jnp.float32),
            pltpu.VMEM((CHUNK,), jnp.float32),
            pltpu.SemaphoreType.DMA,
            pltpu.SemaphoreType.DMA,
        ],
    )
    return k(xwp, rc, ew)


def _tc2_body(p_ref, xwT_ref, dis_ref, b1_ref, w2_ref, hw2_ref):
    dis = dis_ref[...]
    s = (p_ref[0] + p_ref[1] + xwT_ref[...]) * dis[None, :] + b1_ref[...]
    h = jnp.where(s > 0, s, jnp.exp(jnp.minimum(s, 0.0)) - 1.0)
    hw2_ref[...] = jnp.sum(h * w2_ref[...], axis=0) * dis


def _tc2(p, xwT, dis, b1c, W2):
    B = 2048
    grid = NPAD // B
    return pl.pallas_call(
        _tc2_body,
        grid=(grid,),
        in_specs=[
            pl.BlockSpec((NC, D_HID, B), lambda j: (0, 0, j)),
            pl.BlockSpec((D_HID, B), lambda j: (0, j)),
            pl.BlockSpec((B,), lambda j: (j,)),
            pl.BlockSpec((D_HID, 1), lambda j: (0, 0)),
            pl.BlockSpec((D_HID, 1), lambda j: (0, 0)),
        ],
        out_specs=pl.BlockSpec((B,), lambda j: (j,)),
        out_shape=jax.ShapeDtypeStruct((NPAD,), jnp.float32),
    )(p, xwT, dis, b1c, W2)


def _sc_agg1_body(hw2_hbm, rc_hbm, ew_hbm, q_hbm,
                  hv, acc, rcbuf, wbuf, sem):
    cid = lax.axis_index("c")
    sid = lax.axis_index("s")
    wid = cid * NS + sid
    pltpu.async_copy(hw2_hbm, hv, sem)
    pltpu.async_copy(rc_hbm.at[pl.ds(wid * EPT, EPT)], rcbuf, sem)
    pltpu.async_copy(ew_hbm.at[pl.ds(wid * EPT, EPT)], wbuf, sem)
    _zero_vmem(acc, NPAD)
    pltpu.make_async_copy(hw2_hbm, hv, sem).wait()
    pltpu.make_async_copy(rc_hbm.at[pl.ds(0, EPT)], rcbuf, sem).wait()
    pltpu.make_async_copy(ew_hbm.at[pl.ds(0, EPT)], wbuf, sem).wait()

    @plsc.parallel_loop(0, EPT, step=L, unroll=4)
    def body(i):
        v = rcbuf[pl.ds(i, L)]
        r = v & jnp.int32(0xFFFF)
        c = jnp.right_shift(v, 16)
        w = wbuf[pl.ds(i, L)]
        hr = plsc.load_gather(hv, [r])
        plsc.addupdate_scatter(acc, [c], hr * w)

    pltpu.sync_copy(acc, q_hbm.at[wid])


def _sc_agg1(hw2, rc, ew):
    k = pl.kernel(
        _sc_agg1_body,
        out_type=jax.ShapeDtypeStruct((NW, NPAD), jnp.float32),
        mesh=_mesh(),
        compiler_params=pltpu.CompilerParams(needs_layout_passes=False),
        scratch_types=[
            pltpu.VMEM((NPAD,), jnp.float32),
            pltpu.VMEM((NPAD,), jnp.float32),
            pltpu.VMEM((EPT,), jnp.int32),
            pltpu.VMEM((EPT,), jnp.float32),
            pltpu.SemaphoreType.DMA,
        ],
    )
    return k(hw2, rc, ew)


def _tc3_body(q_ref, hw2_ref, dis_ref, b2_ref, out_ref):
    z = (jnp.sum(q_ref[...], axis=0) + hw2_ref[...]) * dis_ref[...] + b2_ref[0, 0]
    out_ref[...] = 1.0 / (1.0 + jnp.exp(-z))


def _tc3(q, hw2, dis, b2c):
    B = 2048
    grid = NPAD // B
    return pl.pallas_call(
        _tc3_body,
        grid=(grid,),
        in_specs=[
            pl.BlockSpec((NW, B), lambda j: (0, j)),
            pl.BlockSpec((B,), lambda j: (j,)),
            pl.BlockSpec((B,), lambda j: (j,)),
            pl.BlockSpec((1, 1), lambda j: (0, 0)),
        ],
        out_specs=pl.BlockSpec((B,), lambda j: (j,)),
        out_shape=jax.ShapeDtypeStruct((NPAD,), jnp.float32),
    )(q, hw2, dis, b2c)


@jax.jit
def kernel(x, edge_index, edge_weight, W1, b1, W2, b2):
    row = edge_index[0]
    col = edge_index[1]
    xp = jnp.zeros((NPAD, D_IN), jnp.float32).at[:N].set(x)
    b1c = b1.reshape(D_HID, 1)
    b2c = b2.reshape(1, 1)

    rc = row | (col << 16)
    degp = _sc_deg(col, edge_weight)
    xwT, xwp, dis = _tc1(xp, W1, degp)
    p = _sc_agg64(xwp, rc, edge_weight)
    hw2 = _tc2(p, xwT, dis, b1c, W2)
    q = _sc_agg1(hw2, rc, edge_weight)
    out = _tc3(q, hw2, dis, b2c)
    return out[:N].reshape(N, 1)
